# Initial kernel scaffold; baseline (speedup 1.0000x reference)
import jax
import jax.numpy as jnp
from jax.experimental import pallas as pl


def kernel(fids_batch, table):
    raise NotImplementedError("write your pallas kernel here")


# speedup vs baseline: 1.4121x; 1.4121x over previous
import functools

import jax
import jax.numpy as jnp
from jax import lax
from jax.experimental import pallas as pl
from jax.experimental.pallas import tpu as pltpu
from jax.experimental.pallas import tpu_sc as plsc

B = 4096
F = 26
EMB = 4
VOCAB = 100000
NC = 2
NS = 16
NW = NC * NS
PB = B // NW
FB = F * PB
L = 16
GROUPS = PB // L
RS_ROWS = VOCAB * EMB // 128
RS_COLS = 128 // EMB


def _rowsum_body(x_ref, o_ref):
    x = x_ref[...]
    ii = lax.broadcasted_iota(jnp.int32, (128, RS_COLS), 0)
    jj = lax.broadcasted_iota(jnp.int32, (128, RS_COLS), 1)
    m = jnp.where(ii // EMB == jj, 1.0, 0.0).astype(jnp.float32)
    o_ref[...] = jnp.dot(x, m, preferred_element_type=jnp.float32)


_rowsum = pl.pallas_call(
    _rowsum_body,
    out_shape=jax.ShapeDtypeStruct((RS_ROWS, RS_COLS), jnp.float32),
)


def _sc_body(idx_hbm, rowsum_hbm, out_hbm, idx_v, vals_v, out_v, sem):
    c = lax.axis_index("c")
    s = lax.axis_index("s")
    wid = s * NC + c
    base = wid * FB

    pltpu.sync_copy(idx_hbm.at[pl.ds(base, FB)], idx_v)

    copies = [
        pltpu.async_copy(
            rowsum_hbm.at[idx_v.at[pl.ds(f * PB, PB)]],
            vals_v.at[pl.ds(f * PB, PB)],
            sem,
        )
        for f in range(F)
    ]
    for cp in copies:
        cp.wait()

    def group(j, carry):
        b0 = j * L
        acc = vals_v[pl.ds(b0, L)]
        for f in range(1, F):
            acc = acc + vals_v[pl.ds(f * PB + b0, L)]
        out_v[pl.ds(b0, L)] = acc
        return carry

    lax.fori_loop(0, GROUPS, group, 0)

    pltpu.sync_copy(out_v, out_hbm.at[pl.ds(wid * PB, PB)])


@functools.partial(
    pl.kernel,
    out_type=jax.ShapeDtypeStruct((B,), jnp.float32),
    mesh=plsc.VectorSubcoreMesh(
        core_axis_name="c", subcore_axis_name="s", num_cores=NC, num_subcores=NS
    ),
    scratch_types=[
        pltpu.VMEM((FB,), jnp.int32),
        pltpu.VMEM((FB,), jnp.float32),
        pltpu.VMEM((PB,), jnp.float32),
        pltpu.SemaphoreType.DMA,
    ],
    compiler_params=pltpu.CompilerParams(use_tc_tiling_on_sc=False),
)
def _sc_kernel(idx_hbm, rowsum_hbm, out_hbm, idx_v, vals_v, out_v, sem):
    _sc_body(idx_hbm, rowsum_hbm, out_hbm, idx_v, vals_v, out_v, sem)


def kernel(fids_batch, table):
    rowsum = _rowsum(table.reshape(RS_ROWS, 128)).reshape(VOCAB)
    idx = (
        fids_batch.astype(jnp.int32)
        .T.reshape(F, NW, PB)
        .transpose(1, 0, 2)
        .reshape(-1)
    )
    return _sc_kernel(idx, rowsum)
